# baseline (speedup 1.0000x reference)
import functools

import jax
import jax.numpy as jnp
from jax import lax
from jax.experimental import pallas as pl
from jax.experimental.pallas import tpu as pltpu
from jax.experimental.pallas import tpu_sc as plsc


def _sc_gather(idx, table, n_chunks, chunk, nc, ns, nbuf):
    nw = nc * ns
    rows_per_w = n_chunks * chunk
    n_total = nw * rows_per_w
    d = table.shape[1]
    ahead = nbuf // 2
    assert nbuf == 2 * ahead and n_chunks >= 2 * nbuf

    mesh = plsc.VectorSubcoreMesh(core_axis_name="c", subcore_axis_name="s")

    @functools.partial(
        pl.kernel,
        out_type=jax.ShapeDtypeStruct((n_total, d), jnp.float32),
        mesh=mesh,
        scratch_types=[
            pltpu.VMEM((n_chunks, chunk), jnp.int32),
            [pltpu.VMEM((chunk, d), jnp.float32)] * nbuf,
            [pltpu.SemaphoreType.DMA] * nbuf,
            [pltpu.SemaphoreType.DMA] * nbuf,
        ],
    )
    def gather_k(idx_hbm, table_hbm, out_hbm, idx_v, bufs, gs, os_):
        wid = lax.axis_index("s") * nc + lax.axis_index("c")
        base = wid * rows_per_w
        pltpu.sync_copy(idx_hbm.at[wid], idx_v)

        def g_copy(j, b):
            return pltpu.make_async_copy(table_hbm.at[idx_v.at[j]], bufs[b], gs[b])

        def o_copy(j, b):
            return pltpu.make_async_copy(
                bufs[b], out_hbm.at[pl.ds(base + j * chunk, chunk)], os_[b])

        def step(j, b):
            g_copy(j, b).wait()
            o_copy(j, b).start()
            b2 = (b + ahead) % nbuf
            o_copy(j - ahead, b2).wait()
            g_copy(j + ahead, b2).start()

        for j in range(ahead):
            g_copy(j, j).start()
        for j in range(ahead):
            g_copy(j, j).wait()
            o_copy(j, j).start()
            g_copy(j + ahead, j + ahead).start()

        q, r = divmod(n_chunks - 2 * ahead, nbuf)

        def body(p, carry):
            j0 = nbuf * p + ahead
            for i in range(nbuf):
                step(j0 + i, (ahead + i) % nbuf)
            return carry

        lax.fori_loop(0, q, body, 0)
        for j in range(n_chunks - ahead - r, n_chunks - ahead):
            step(j, j % nbuf)
        for j in range(n_chunks - ahead, n_chunks):
            b = j % nbuf
            g_copy(j, b).wait()
            o_copy(j, b).start()
            o_copy(j - ahead, (b + ahead) % nbuf).wait()
        for j in range(n_chunks - ahead, n_chunks):
            o_copy(j, j % nbuf).wait()

    return gather_k(idx, table)


def kernel(indices, table):
    b0, b1 = indices.shape
    v, d = table.shape
    n = b0 * b1

    info = plsc.get_sparse_core_info()
    nc, ns = info.num_cores, info.num_subcores
    nw = nc * ns

    chunk = 16
    nbuf = 6
    per_w = n // nw
    n_chunks = per_w // chunk
    assert n == nw * n_chunks * chunk, (n, nw, chunk)

    idx = indices.T.reshape(nw, n_chunks, chunk).astype(jnp.int32)
    out = _sc_gather(idx, table, n_chunks, chunk, nc, ns, nbuf)
    return out.reshape(b1, b0, d).transpose(1, 0, 2)
